# Initial kernel scaffold; baseline (speedup 1.0000x reference)
import jax
import jax.numpy as jnp
from jax.experimental import pallas as pl


def kernel(patches, pos_table):
    raise NotImplementedError("write your pallas kernel here")


# speedup vs baseline: 1.0339x; 1.0339x over previous
import jax
import jax.numpy as jnp
from jax.experimental import pallas as pl


def _add_block(patches_ref, pos_ref, out_ref):
    out_ref[...] = patches_ref[...] + pos_ref[...]


def kernel(patches, pos_table):
    batch, n_patches, model_dim = patches.shape
    bb = 4
    return pl.pallas_call(
        _add_block,
        grid=(batch // bb,),
        in_specs=[
            pl.BlockSpec((bb, n_patches, model_dim), lambda i: (i, 0, 0)),
            pl.BlockSpec((n_patches, model_dim), lambda i: (0, 0)),
        ],
        out_specs=pl.BlockSpec((bb, n_patches, model_dim), lambda i: (i, 0, 0)),
        out_shape=jax.ShapeDtypeStruct((batch, n_patches, model_dim), patches.dtype),
    )(patches, pos_table)
